# Initial kernel scaffold; baseline (speedup 1.0000x reference)
import jax
import jax.numpy as jnp
from jax.experimental import pallas as pl


def kernel(adj_norm, adj_orig, features, W_base, W_mean, W1, b1, W2, b2):
    raise NotImplementedError("write your pallas kernel here")


# speedup vs baseline: 1.8498x; 1.8498x over previous
import jax
import jax.numpy as jnp
from jax.experimental import pallas as pl

N = 4096
D = 256
H = 128
EMB = 64
C = 40
TEMP = 1.0
EPS = 1e-06

BM = 512
BS = 512


def _mm_kernel(x_ref, w_ref, o_ref):
    o_ref[...] = jnp.dot(x_ref[...], w_ref[...])


def _mm(x, w):
    m, k = x.shape
    _, n = w.shape
    return pl.pallas_call(
        _mm_kernel,
        out_shape=jax.ShapeDtypeStruct((m, n), jnp.float32),
    )(x, w)


def _spmm_kernel_relu(a_ref, x_ref, o_ref):
    o_ref[...] = jnp.maximum(jnp.dot(a_ref[...], x_ref[...]), 0.0)


def _spmm_kernel(a_ref, x_ref, o_ref):
    o_ref[...] = jnp.dot(a_ref[...], x_ref[...])


def _spmm(a, x, relu=False):
    n, k = x.shape
    body = _spmm_kernel_relu if relu else _spmm_kernel
    return pl.pallas_call(
        body,
        grid=(N // BM,),
        in_specs=[
            pl.BlockSpec((BM, N), lambda i: (i, 0)),
            pl.BlockSpec((N, k), lambda i: (0, 0)),
        ],
        out_specs=pl.BlockSpec((BM, k), lambda i: (i, 0)),
        out_shape=jax.ShapeDtypeStruct((N, k), jnp.float32),
    )(a, x)


def _zmax_kernel(zi_ref, z_ref, m_ref):
    l = jax.lax.dot_general(zi_ref[...], z_ref[...], (((1,), (1,)), ((), ())))
    bm = jnp.max(l).reshape(1, 1)

    @pl.when(pl.program_id(0) == 0)
    def _():
        m_ref[...] = bm

    @pl.when(pl.program_id(0) > 0)
    def _():
        m_ref[...] = jnp.maximum(m_ref[...], bm)


def _zmax(z):
    return pl.pallas_call(
        _zmax_kernel,
        grid=(N // BM,),
        in_specs=[
            pl.BlockSpec((BM, EMB), lambda i: (i, 0)),
            pl.BlockSpec((N, EMB), lambda i: (0, 0)),
        ],
        out_specs=pl.BlockSpec((1, 1), lambda i: (0, 0)),
        out_shape=jax.ShapeDtypeStruct((1, 1), jnp.float32),
    )(z, z)


def _sample_kernel(zi_ref, zj_ref, u_ref, gmax_ref,
                   logits_ref, adj_ref, dinv_ref):
    i = pl.program_id(0)
    j = pl.program_id(1)
    nb = pl.num_programs(1)
    l = jax.lax.dot_general(zi_ref[...], zj_ref[...], (((1,), (1,)), ((), ())))
    logits_ref[...] = l
    gmax = gmax_ref[0, 0]
    p = jnp.clip(l / gmax, EPS, 1.0 - EPS)
    lp = jnp.log(p) - jnp.log1p(-p)
    rows = i * BS + jax.lax.broadcasted_iota(jnp.int32, (BS, BS), 0)
    cols = j * BS + jax.lax.broadcasted_iota(jnp.int32, (BS, BS), 1)
    u = u_ref[...]
    u_up = jnp.where(rows < cols, u, u.T)
    ln = jnp.log(u_up) - jnp.log1p(-u_up)
    soft = jax.nn.sigmoid((lp + ln) / TEMP)
    hard = jnp.round(soft)
    a = jnp.where(rows == cols, 0.0, hard)
    adj_ref[...] = a.astype(jnp.int8)
    rs = jnp.sum(a, axis=1, keepdims=True)

    @pl.when(j == 0)
    def _():
        dinv_ref[...] = rs + 1.0

    @pl.when(j > 0)
    def _():
        dinv_ref[...] = dinv_ref[...] + rs

    @pl.when(j == nb - 1)
    def _():
        dinv_ref[...] = 1.0 / jnp.sqrt(dinv_ref[...])


def _sample(z, u, gmax):
    nb = N // BS
    return pl.pallas_call(
        _sample_kernel,
        grid=(nb, nb),
        in_specs=[
            pl.BlockSpec((BS, EMB), lambda i, j: (i, 0)),
            pl.BlockSpec((BS, EMB), lambda i, j: (j, 0)),
            pl.BlockSpec((BS, BS),
                         lambda i, j: (jnp.minimum(i, j), jnp.maximum(i, j))),
            pl.BlockSpec((1, 1), lambda i, j: (0, 0)),
        ],
        out_specs=[
            pl.BlockSpec((BS, BS), lambda i, j: (i, j)),
            pl.BlockSpec((BS, BS), lambda i, j: (i, j)),
            pl.BlockSpec((BS, 1), lambda i, j: (i, 0)),
        ],
        out_shape=[
            jax.ShapeDtypeStruct((N, N), jnp.float32),
            jax.ShapeDtypeStruct((N, N), jnp.int8),
            jax.ShapeDtypeStruct((N, 1), jnp.float32),
        ],
    )(z, z, u, gmax)


def _h_kernel(a_ref, x1_ref, dinv_ref, b1_ref, w2_ref, yg_ref):
    i = pl.program_id(0)
    dinv = dinv_ref[...]
    y1 = dinv * x1_ref[...]
    a = a_ref[...].astype(jnp.float32)
    acc = jnp.dot(a, y1)
    dinv_i = dinv_ref[pl.ds(i * BM, BM), :]
    y1_i = dinv_i * x1_ref[pl.ds(i * BM, BM), :]
    h = jnp.maximum(dinv_i * (acc + y1_i) + b1_ref[...], 0.0)
    yg_ref[...] = dinv_i * jnp.dot(h, w2_ref[...])


def _h_pass(adj_new, x1, dinv, b1, w2):
    return pl.pallas_call(
        _h_kernel,
        grid=(N // BM,),
        in_specs=[
            pl.BlockSpec((BM, N), lambda i: (i, 0)),
            pl.BlockSpec((N, H), lambda i: (0, 0)),
            pl.BlockSpec((N, 1), lambda i: (0, 0)),
            pl.BlockSpec((1, H), lambda i: (0, 0)),
            pl.BlockSpec((H, C), lambda i: (0, 0)),
        ],
        out_specs=pl.BlockSpec((BM, C), lambda i: (i, 0)),
        out_shape=jax.ShapeDtypeStruct((N, C), jnp.float32),
    )(adj_new, x1, dinv, b1, w2)


def _nc_kernel(a_ref, yg_ref, dinv_ref, b2_ref, o_ref):
    i = pl.program_id(0)
    a = a_ref[...].astype(jnp.float32)
    acc = jnp.dot(a, yg_ref[...])
    yg_i = yg_ref[pl.ds(i * BM, BM), :]
    dinv_i = dinv_ref[pl.ds(i * BM, BM), :]
    o_ref[...] = dinv_i * (acc + yg_i) + b2_ref[...]


def _nc_pass(adj_new, yg, dinv, b2):
    return pl.pallas_call(
        _nc_kernel,
        grid=(N // BM,),
        in_specs=[
            pl.BlockSpec((BM, N), lambda i: (i, 0)),
            pl.BlockSpec((N, C), lambda i: (0, 0)),
            pl.BlockSpec((N, 1), lambda i: (0, 0)),
            pl.BlockSpec((1, C), lambda i: (0, 0)),
        ],
        out_specs=pl.BlockSpec((BM, C), lambda i: (i, 0)),
        out_shape=jax.ShapeDtypeStruct((N, C), jnp.float32),
    )(adj_new, yg, dinv, b2)


def kernel(adj_norm, adj_orig, features, W_base, W_mean, W1, b1, W2, b2):
    u = jax.random.uniform(jax.random.key(1234), (N, N),
                           minval=EPS, maxval=1.0 - EPS)
    xw = _mm(features, jnp.concatenate([W_base, W1], axis=1))
    xb, x1 = xw[:, :H], xw[:, H:]
    hidden = _spmm(adj_norm, xb)
    m = _mm(hidden, W_mean)
    z = _spmm(adj_norm, m, relu=True)
    gmax = _zmax(z)
    adj_logits, adj_new, dinv = _sample(z, u, gmax)
    yg = _h_pass(adj_new, x1, dinv, b1.reshape(1, H), W2)
    nc_logits = _nc_pass(adj_new, yg, dinv, b2.reshape(1, C))
    return (nc_logits, adj_logits)


# speedup vs baseline: 1.8536x; 1.0020x over previous
import jax
import jax.numpy as jnp
from jax.experimental import pallas as pl

N = 4096
D = 256
H = 128
EMB = 64
C = 40
TEMP = 1.0
EPS = 1e-06

BM = 512
BS = 512


def _mm_kernel(x_ref, w_ref, o_ref):
    o_ref[...] = jnp.dot(x_ref[...], w_ref[...])


def _mm(x, w):
    m, k = x.shape
    _, n = w.shape
    return pl.pallas_call(
        _mm_kernel,
        out_shape=jax.ShapeDtypeStruct((m, n), jnp.float32),
    )(x, w)


def _spmm_kernel_relu(a_ref, x_ref, o_ref):
    o_ref[...] = jnp.maximum(jnp.dot(a_ref[...], x_ref[...]), 0.0)


def _spmm_kernel(a_ref, x_ref, o_ref):
    o_ref[...] = jnp.dot(a_ref[...], x_ref[...])


def _spmm(a, x, relu=False):
    n, k = x.shape
    body = _spmm_kernel_relu if relu else _spmm_kernel
    return pl.pallas_call(
        body,
        grid=(N // BM,),
        in_specs=[
            pl.BlockSpec((BM, N), lambda i: (i, 0)),
            pl.BlockSpec((N, k), lambda i: (0, 0)),
        ],
        out_specs=pl.BlockSpec((BM, k), lambda i: (i, 0)),
        out_shape=jax.ShapeDtypeStruct((N, k), jnp.float32),
    )(a, x)


def _zmax_kernel(zi_ref, z_ref, m_ref):
    l = jax.lax.dot_general(zi_ref[...], z_ref[...], (((1,), (1,)), ((), ())))
    bm = jnp.max(l).reshape(1, 1)

    @pl.when(pl.program_id(0) == 0)
    def _():
        m_ref[...] = bm

    @pl.when(pl.program_id(0) > 0)
    def _():
        m_ref[...] = jnp.maximum(m_ref[...], bm)


def _zmax(z):
    return pl.pallas_call(
        _zmax_kernel,
        grid=(N // BM,),
        in_specs=[
            pl.BlockSpec((BM, EMB), lambda i: (i, 0)),
            pl.BlockSpec((N, EMB), lambda i: (0, 0)),
        ],
        out_specs=pl.BlockSpec((1, 1), lambda i: (0, 0)),
        out_shape=jax.ShapeDtypeStruct((1, 1), jnp.float32),
    )(z, z)


def _sample_kernel(zi_ref, zj_ref, u_ref, gmax_ref,
                   logits_ref, adj_ref, dinv_ref):
    i = pl.program_id(0)
    j = pl.program_id(1)
    nb = pl.num_programs(1)
    l = jax.lax.dot_general(zi_ref[...], zj_ref[...], (((1,), (1,)), ((), ())))
    logits_ref[...] = l
    gmax = gmax_ref[0, 0]
    p = jnp.clip(l / gmax, EPS, 1.0 - EPS)
    lp = jnp.log(p) - jnp.log1p(-p)
    rows = i * BS + jax.lax.broadcasted_iota(jnp.int32, (BS, BS), 0)
    cols = j * BS + jax.lax.broadcasted_iota(jnp.int32, (BS, BS), 1)
    u = u_ref[...]
    u_up = jnp.where(rows < cols, u, u.T)
    ln = jnp.log(u_up) - jnp.log1p(-u_up)
    soft = jax.nn.sigmoid((lp + ln) / TEMP)
    hard = jnp.round(soft)
    a = jnp.where(rows == cols, 0.0, hard)
    adj_ref[...] = a.astype(jnp.int8)
    rs = jnp.sum(a, axis=1, keepdims=True)

    @pl.when(j == 0)
    def _():
        dinv_ref[...] = rs + 1.0

    @pl.when(j > 0)
    def _():
        dinv_ref[...] = dinv_ref[...] + rs

    @pl.when(j == nb - 1)
    def _():
        dinv_ref[...] = 1.0 / jnp.sqrt(dinv_ref[...])


def _sample(z, u, gmax):
    nb = N // BS
    return pl.pallas_call(
        _sample_kernel,
        grid=(nb, nb),
        in_specs=[
            pl.BlockSpec((BS, EMB), lambda i, j: (i, 0)),
            pl.BlockSpec((BS, EMB), lambda i, j: (j, 0)),
            pl.BlockSpec((BS, BS),
                         lambda i, j: (jnp.minimum(i, j), jnp.maximum(i, j))),
            pl.BlockSpec((1, 1), lambda i, j: (0, 0)),
        ],
        out_specs=[
            pl.BlockSpec((BS, BS), lambda i, j: (i, j)),
            pl.BlockSpec((BS, BS), lambda i, j: (i, j)),
            pl.BlockSpec((BS, 1), lambda i, j: (i, 0)),
        ],
        out_shape=[
            jax.ShapeDtypeStruct((N, N), jnp.float32),
            jax.ShapeDtypeStruct((N, N), jnp.int8),
            jax.ShapeDtypeStruct((N, 1), jnp.float32),
        ],
    )(z, z, u, gmax)


def _h_kernel(a_ref, x1_ref, dinv_ref, b1_ref, w2_ref, yg_ref):
    i = pl.program_id(0)
    dinv = dinv_ref[...]
    y1 = dinv * x1_ref[...]
    a = a_ref[...].astype(jnp.bfloat16)
    acc = jnp.dot(a, y1.astype(jnp.bfloat16),
                  preferred_element_type=jnp.float32)
    dinv_i = dinv_ref[pl.ds(i * BM, BM), :]
    y1_i = dinv_i * x1_ref[pl.ds(i * BM, BM), :]
    h = jnp.maximum(dinv_i * (acc + y1_i) + b1_ref[...], 0.0)
    yg_ref[...] = dinv_i * jnp.dot(h, w2_ref[...])


def _h_pass(adj_new, x1, dinv, b1, w2):
    return pl.pallas_call(
        _h_kernel,
        grid=(N // BM,),
        in_specs=[
            pl.BlockSpec((BM, N), lambda i: (i, 0)),
            pl.BlockSpec((N, H), lambda i: (0, 0)),
            pl.BlockSpec((N, 1), lambda i: (0, 0)),
            pl.BlockSpec((1, H), lambda i: (0, 0)),
            pl.BlockSpec((H, C), lambda i: (0, 0)),
        ],
        out_specs=pl.BlockSpec((BM, C), lambda i: (i, 0)),
        out_shape=jax.ShapeDtypeStruct((N, C), jnp.float32),
    )(adj_new, x1, dinv, b1, w2)


def _nc_kernel(a_ref, yg_ref, dinv_ref, b2_ref, o_ref):
    i = pl.program_id(0)
    a = a_ref[...].astype(jnp.bfloat16)
    acc = jnp.dot(a, yg_ref[...].astype(jnp.bfloat16),
                  preferred_element_type=jnp.float32)
    yg_i = yg_ref[pl.ds(i * BM, BM), :]
    dinv_i = dinv_ref[pl.ds(i * BM, BM), :]
    o_ref[...] = dinv_i * (acc + yg_i) + b2_ref[...]


def _nc_pass(adj_new, yg, dinv, b2):
    return pl.pallas_call(
        _nc_kernel,
        grid=(N // BM,),
        in_specs=[
            pl.BlockSpec((BM, N), lambda i: (i, 0)),
            pl.BlockSpec((N, C), lambda i: (0, 0)),
            pl.BlockSpec((N, 1), lambda i: (0, 0)),
            pl.BlockSpec((1, C), lambda i: (0, 0)),
        ],
        out_specs=pl.BlockSpec((BM, C), lambda i: (i, 0)),
        out_shape=jax.ShapeDtypeStruct((N, C), jnp.float32),
    )(adj_new, yg, dinv, b2)


def kernel(adj_norm, adj_orig, features, W_base, W_mean, W1, b1, W2, b2):
    u = jax.random.uniform(jax.random.key(1234), (N, N),
                           minval=EPS, maxval=1.0 - EPS)
    xw = _mm(features, jnp.concatenate([W_base, W1], axis=1))
    xb, x1 = xw[:, :H], xw[:, H:]
    hidden = _spmm(adj_norm, xb)
    m = _mm(hidden, W_mean)
    z = _spmm(adj_norm, m, relu=True)
    gmax = _zmax(z)
    adj_logits, adj_new, dinv = _sample(z, u, gmax)
    yg = _h_pass(adj_new, x1, dinv, b1.reshape(1, H), W2)
    nc_logits = _nc_pass(adj_new, yg, dinv, b2.reshape(1, C))
    return (nc_logits, adj_logits)


# speedup vs baseline: 2.0089x; 1.0838x over previous
import jax
import jax.numpy as jnp
from jax.experimental import pallas as pl

N = 4096
D = 256
H = 128
EMB = 64
C = 40
TEMP = 1.0
EPS = 1e-06

BM = 512
BS = 1024


def _mm_kernel(x_ref, w_ref, o_ref):
    o_ref[...] = jnp.dot(x_ref[...], w_ref[...])


def _mm(x, w):
    m, k = x.shape
    _, n = w.shape
    return pl.pallas_call(
        _mm_kernel,
        out_shape=jax.ShapeDtypeStruct((m, n), jnp.float32),
    )(x, w)


def _spmm_kernel(a_ref, x_ref, o_ref):
    o_ref[...] = jnp.dot(a_ref[...], x_ref[...])


def _spmm(a, x):
    n, k = x.shape
    return pl.pallas_call(
        _spmm_kernel,
        grid=(N // BM,),
        in_specs=[
            pl.BlockSpec((BM, N), lambda i: (i, 0)),
            pl.BlockSpec((N, k), lambda i: (0, 0)),
        ],
        out_specs=pl.BlockSpec((BM, k), lambda i: (i, 0)),
        out_shape=jax.ShapeDtypeStruct((N, k), jnp.float32),
    )(a, x)


def _z_kernel(a_ref, x_ref, o_ref, m_ref):
    z = jnp.maximum(jnp.dot(a_ref[...], x_ref[...]), 0.0)
    o_ref[...] = z
    bm = jnp.max(jnp.sum(z * z, axis=1)).reshape(1, 1)

    @pl.when(pl.program_id(0) == 0)
    def _():
        m_ref[...] = bm

    @pl.when(pl.program_id(0) > 0)
    def _():
        m_ref[...] = jnp.maximum(m_ref[...], bm)


def _z_pass(a, x):
    return pl.pallas_call(
        _z_kernel,
        grid=(N // BM,),
        in_specs=[
            pl.BlockSpec((BM, N), lambda i: (i, 0)),
            pl.BlockSpec((N, EMB), lambda i: (0, 0)),
        ],
        out_specs=[
            pl.BlockSpec((BM, EMB), lambda i: (i, 0)),
            pl.BlockSpec((1, 1), lambda i: (0, 0)),
        ],
        out_shape=[
            jax.ShapeDtypeStruct((N, EMB), jnp.float32),
            jax.ShapeDtypeStruct((1, 1), jnp.float32),
        ],
    )(a, x)


def _sample_kernel(zi_ref, zj_ref, u_ref, gmax_ref,
                   logits_ref, adj_ref, dinv_ref):
    i = pl.program_id(0)
    j = pl.program_id(1)
    nb = pl.num_programs(1)
    l = jax.lax.dot_general(zi_ref[...], zj_ref[...], (((1,), (1,)), ((), ())))
    logits_ref[...] = l
    gmax = gmax_ref[0, 0]
    p = jnp.clip(l / gmax, EPS, 1.0 - EPS)
    lp = jnp.log(p) - jnp.log1p(-p)
    rows = i * BS + jax.lax.broadcasted_iota(jnp.int32, (BS, BS), 0)
    cols = j * BS + jax.lax.broadcasted_iota(jnp.int32, (BS, BS), 1)
    u = u_ref[...]
    u_up = jnp.where(rows < cols, u, u.T)
    ln = jnp.log(u_up) - jnp.log1p(-u_up)
    soft = jax.nn.sigmoid((lp + ln) / TEMP)
    hard = jnp.round(soft)
    a = jnp.where(rows == cols, 0.0, hard)
    adj_ref[...] = a.astype(jnp.int8)
    rs = jnp.sum(a, axis=1, keepdims=True)

    @pl.when(j == 0)
    def _():
        dinv_ref[...] = rs + 1.0

    @pl.when(j > 0)
    def _():
        dinv_ref[...] = dinv_ref[...] + rs

    @pl.when(j == nb - 1)
    def _():
        dinv_ref[...] = 1.0 / jnp.sqrt(dinv_ref[...])


def _sample(z, u, gmax):
    nb = N // BS
    return pl.pallas_call(
        _sample_kernel,
        grid=(nb, nb),
        in_specs=[
            pl.BlockSpec((BS, EMB), lambda i, j: (i, 0)),
            pl.BlockSpec((BS, EMB), lambda i, j: (j, 0)),
            pl.BlockSpec((BS, BS),
                         lambda i, j: (jnp.minimum(i, j), jnp.maximum(i, j))),
            pl.BlockSpec((1, 1), lambda i, j: (0, 0)),
        ],
        out_specs=[
            pl.BlockSpec((BS, BS), lambda i, j: (i, j)),
            pl.BlockSpec((BS, BS), lambda i, j: (i, j)),
            pl.BlockSpec((BS, 1), lambda i, j: (i, 0)),
        ],
        out_shape=[
            jax.ShapeDtypeStruct((N, N), jnp.float32),
            jax.ShapeDtypeStruct((N, N), jnp.int8),
            jax.ShapeDtypeStruct((N, 1), jnp.float32),
        ],
    )(z, z, u, gmax)


def _h_kernel(a_ref, x1_ref, dinv_ref, b1_ref, w2_ref, yg_ref):
    i = pl.program_id(0)
    dinv = dinv_ref[...]
    y1 = dinv * x1_ref[...]
    a = a_ref[...].astype(jnp.bfloat16)
    acc = jnp.dot(a, y1.astype(jnp.bfloat16),
                  preferred_element_type=jnp.float32)
    dinv_i = dinv_ref[pl.ds(i * BM, BM), :]
    y1_i = dinv_i * x1_ref[pl.ds(i * BM, BM), :]
    h = jnp.maximum(dinv_i * (acc + y1_i) + b1_ref[...], 0.0)
    yg_ref[...] = dinv_i * jnp.dot(h, w2_ref[...])


def _h_pass(adj_new, x1, dinv, b1, w2):
    return pl.pallas_call(
        _h_kernel,
        grid=(N // BM,),
        in_specs=[
            pl.BlockSpec((BM, N), lambda i: (i, 0)),
            pl.BlockSpec((N, H), lambda i: (0, 0)),
            pl.BlockSpec((N, 1), lambda i: (0, 0)),
            pl.BlockSpec((1, H), lambda i: (0, 0)),
            pl.BlockSpec((H, C), lambda i: (0, 0)),
        ],
        out_specs=pl.BlockSpec((BM, C), lambda i: (i, 0)),
        out_shape=jax.ShapeDtypeStruct((N, C), jnp.float32),
    )(adj_new, x1, dinv, b1, w2)


def _nc_kernel(a_ref, yg_ref, dinv_ref, b2_ref, o_ref):
    i = pl.program_id(0)
    a = a_ref[...].astype(jnp.bfloat16)
    acc = jnp.dot(a, yg_ref[...].astype(jnp.bfloat16),
                  preferred_element_type=jnp.float32)
    yg_i = yg_ref[pl.ds(i * BM, BM), :]
    dinv_i = dinv_ref[pl.ds(i * BM, BM), :]
    o_ref[...] = dinv_i * (acc + yg_i) + b2_ref[...]


def _nc_pass(adj_new, yg, dinv, b2):
    return pl.pallas_call(
        _nc_kernel,
        grid=(N // BM,),
        in_specs=[
            pl.BlockSpec((BM, N), lambda i: (i, 0)),
            pl.BlockSpec((N, C), lambda i: (0, 0)),
            pl.BlockSpec((N, 1), lambda i: (0, 0)),
            pl.BlockSpec((1, C), lambda i: (0, 0)),
        ],
        out_specs=pl.BlockSpec((BM, C), lambda i: (i, 0)),
        out_shape=jax.ShapeDtypeStruct((N, C), jnp.float32),
    )(adj_new, yg, dinv, b2)


def kernel(adj_norm, adj_orig, features, W_base, W_mean, W1, b1, W2, b2):
    u = jax.random.uniform(jax.random.key(1234), (N, N),
                           minval=EPS, maxval=1.0 - EPS)
    xw = _mm(features, jnp.concatenate([W_base, W1], axis=1))
    xb, x1 = xw[:, :H], xw[:, H:]
    hidden = _spmm(adj_norm, xb)
    m = _mm(hidden, W_mean)
    z, gmax = _z_pass(adj_norm, m)
    adj_logits, adj_new, dinv = _sample(z, u, gmax)
    yg = _h_pass(adj_new, x1, dinv, b1.reshape(1, H), W2)
    nc_logits = _nc_pass(adj_new, yg, dinv, b2.reshape(1, C))
    return (nc_logits, adj_logits)


# speedup vs baseline: 2.0292x; 1.0101x over previous
import jax
import jax.numpy as jnp
import numpy as np
from jax.experimental import pallas as pl
from jax.experimental.pallas import tpu as pltpu

N = 4096
D = 256
H = 128
EMB = 64
C = 40
TEMP = 1.0
EPS = 1e-06

BM = 512
BS = 1024
NB = N // BS

_ORDER = []
for _i in range(NB):
    _ORDER.append((_i, _i))
    for _j in range(_i + 1, NB):
        _ORDER.append((_i, _j))
        _ORDER.append((_j, _i))
_OI = np.array([p[0] for p in _ORDER], dtype=np.int32)
_OJ = np.array([p[1] for p in _ORDER], dtype=np.int32)


def _hidden_pass(features, wcat, adj):
    return pl.pallas_call(
        _hidden_kernel,
        grid=(N // BM,),
        in_specs=[
            pl.BlockSpec((N, 2 * H), lambda i: (0, 0)),
            pl.BlockSpec((D, 2 * H), lambda i: (0, 0)),
            pl.BlockSpec((BM, N), lambda i: (i, 0)),
        ],
        out_specs=[
            pl.BlockSpec((BM, H), lambda i: (i, 0)),
            pl.BlockSpec((N, H), lambda i: (0, 0)),
        ],
        out_shape=[
            jax.ShapeDtypeStruct((N, H), jnp.float32),
            jax.ShapeDtypeStruct((N, H), jnp.float32),
        ],
        scratch_shapes=[pltpu.VMEM((N, 2 * H), jnp.float32)],
    )(features, wcat, adj)


def _hidden_kernel(feat_ref, wcat_ref, a_ref, hid_ref, x1_ref, xw_ref):
    @pl.when(pl.program_id(0) == 0)
    def _():
        xw_ref[...] = jnp.dot(feat_ref[...], wcat_ref[...])
        x1_ref[...] = xw_ref[:, H:]

    hid_ref[...] = jnp.dot(a_ref[...], xw_ref[:, :H])


def _z_kernel(hid_ref, wm_ref, a_ref, z_ref, gmax_ref, m_ref):
    @pl.when(pl.program_id(0) == 0)
    def _():
        m_ref[...] = jnp.dot(hid_ref[...], wm_ref[...])

    z = jnp.maximum(jnp.dot(a_ref[...], m_ref[...]), 0.0)
    z_ref[...] = z
    bm = jnp.max(jnp.sum(z * z, axis=1)).reshape(1, 1)

    @pl.when(pl.program_id(0) == 0)
    def _():
        gmax_ref[...] = bm

    @pl.when(pl.program_id(0) > 0)
    def _():
        gmax_ref[...] = jnp.maximum(gmax_ref[...], bm)


def _z_pass(hidden, wm, adj):
    return pl.pallas_call(
        _z_kernel,
        grid=(N // BM,),
        in_specs=[
            pl.BlockSpec((N, H), lambda i: (0, 0)),
            pl.BlockSpec((H, EMB), lambda i: (0, 0)),
            pl.BlockSpec((BM, N), lambda i: (i, 0)),
        ],
        out_specs=[
            pl.BlockSpec((BM, EMB), lambda i: (i, 0)),
            pl.BlockSpec((1, 1), lambda i: (0, 0)),
        ],
        out_shape=[
            jax.ShapeDtypeStruct((N, EMB), jnp.float32),
            jax.ShapeDtypeStruct((1, 1), jnp.float32),
        ],
        scratch_shapes=[pltpu.VMEM((N, EMB), jnp.float32)],
    )(hidden, wm, adj)


def _sample_kernel(oi_ref, oj_ref, zi_ref, zj_ref, u_ref, gmax_ref,
                   logits_ref, adj_ref, degp_ref):
    t = pl.program_id(0)
    i = oi_ref[t]
    j = oj_ref[t]
    l = jax.lax.dot_general(zi_ref[...], zj_ref[...], (((1,), (1,)), ((), ())))
    logits_ref[...] = l
    gmax = gmax_ref[0, 0]
    p = jnp.clip(l / gmax, EPS, 1.0 - EPS)
    lp = jnp.log(p) - jnp.log1p(-p)
    rows = i * BS + jax.lax.broadcasted_iota(jnp.int32, (BS, BS), 0)
    cols = j * BS + jax.lax.broadcasted_iota(jnp.int32, (BS, BS), 1)
    u = u_ref[...]
    u_up = jnp.where(rows < cols, u, u.T)
    ln = jnp.log(u_up) - jnp.log1p(-u_up)
    soft = jax.nn.sigmoid((lp + ln) / TEMP)
    hard = jnp.round(soft)
    a = jnp.where(rows == cols, 0.0, hard)
    adj_ref[...] = a.astype(jnp.int8)
    degp_ref[...] = jnp.sum(a, axis=1, keepdims=True).reshape(1, BS, 1)


def _sample(z, u, gmax):
    grid_spec = pltpu.PrefetchScalarGridSpec(
        num_scalar_prefetch=2,
        grid=(len(_ORDER),),
        in_specs=[
            pl.BlockSpec((BS, EMB), lambda t, oi, oj: (oi[t], 0)),
            pl.BlockSpec((BS, EMB), lambda t, oi, oj: (oj[t], 0)),
            pl.BlockSpec((BS, BS),
                         lambda t, oi, oj: (jnp.minimum(oi[t], oj[t]),
                                            jnp.maximum(oi[t], oj[t]))),
            pl.BlockSpec((1, 1), lambda t, oi, oj: (0, 0)),
        ],
        out_specs=[
            pl.BlockSpec((BS, BS), lambda t, oi, oj: (oi[t], oj[t])),
            pl.BlockSpec((BS, BS), lambda t, oi, oj: (oi[t], oj[t])),
            pl.BlockSpec((1, BS, 1), lambda t, oi, oj: (oj[t], oi[t], 0)),
        ],
    )
    return pl.pallas_call(
        _sample_kernel,
        grid_spec=grid_spec,
        out_shape=[
            jax.ShapeDtypeStruct((N, N), jnp.float32),
            jax.ShapeDtypeStruct((N, N), jnp.int8),
            jax.ShapeDtypeStruct((NB, N, 1), jnp.float32),
        ],
    )(jnp.asarray(_OI), jnp.asarray(_OJ), z, z, u, gmax)


def _h_kernel(a_ref, x1_ref, degp_ref, b1_ref, w2_ref, yg_ref):
    i = pl.program_id(0)
    deg = jnp.sum(degp_ref[...], axis=0) + 1.0
    dinv = 1.0 / jnp.sqrt(deg)
    y1 = dinv * x1_ref[...]
    a = a_ref[...].astype(jnp.bfloat16)
    acc = jnp.dot(a, y1.astype(jnp.bfloat16),
                  preferred_element_type=jnp.float32)
    deg_i = jnp.sum(degp_ref[:, pl.ds(i * BM, BM), :], axis=0) + 1.0
    dinv_i = 1.0 / jnp.sqrt(deg_i)
    y1_i = dinv_i * x1_ref[pl.ds(i * BM, BM), :]
    h = jnp.maximum(dinv_i * (acc + y1_i) + b1_ref[...], 0.0)
    yg_ref[...] = dinv_i * jnp.dot(h, w2_ref[...])


def _h_pass(adj_new, x1, degp, b1, w2):
    return pl.pallas_call(
        _h_kernel,
        grid=(N // BM,),
        in_specs=[
            pl.BlockSpec((BM, N), lambda i: (i, 0)),
            pl.BlockSpec((N, H), lambda i: (0, 0)),
            pl.BlockSpec((NB, N, 1), lambda i: (0, 0, 0)),
            pl.BlockSpec((1, H), lambda i: (0, 0)),
            pl.BlockSpec((H, C), lambda i: (0, 0)),
        ],
        out_specs=pl.BlockSpec((BM, C), lambda i: (i, 0)),
        out_shape=jax.ShapeDtypeStruct((N, C), jnp.float32),
    )(adj_new, x1, degp, b1, w2)


def _nc_kernel(a_ref, yg_ref, degp_ref, b2_ref, o_ref):
    i = pl.program_id(0)
    deg = jnp.sum(degp_ref[...], axis=0) + 1.0
    dinv = 1.0 / jnp.sqrt(deg)
    a = a_ref[...].astype(jnp.bfloat16)
    acc = jnp.dot(a, yg_ref[...].astype(jnp.bfloat16),
                  preferred_element_type=jnp.float32)
    yg_i = yg_ref[pl.ds(i * BM, BM), :]
    deg_i = jnp.sum(degp_ref[:, pl.ds(i * BM, BM), :], axis=0) + 1.0
    dinv_i = 1.0 / jnp.sqrt(deg_i)
    o_ref[...] = dinv_i * (acc + yg_i) + b2_ref[...]


def _nc_pass(adj_new, yg, degp, b2):
    return pl.pallas_call(
        _nc_kernel,
        grid=(N // BM,),
        in_specs=[
            pl.BlockSpec((BM, N), lambda i: (i, 0)),
            pl.BlockSpec((N, C), lambda i: (0, 0)),
            pl.BlockSpec((NB, N, 1), lambda i: (0, 0, 0)),
            pl.BlockSpec((1, C), lambda i: (0, 0)),
        ],
        out_specs=pl.BlockSpec((BM, C), lambda i: (i, 0)),
        out_shape=jax.ShapeDtypeStruct((N, C), jnp.float32),
    )(adj_new, yg, degp, b2)


def kernel(adj_norm, adj_orig, features, W_base, W_mean, W1, b1, W2, b2):
    u = jax.random.uniform(jax.random.key(1234), (N, N),
                           minval=EPS, maxval=1.0 - EPS)
    wcat = jnp.concatenate([W_base, W1], axis=1)
    hidden, x1 = _hidden_pass(features, wcat, adj_norm)
    z, gmax = _z_pass(hidden, W_mean, adj_norm)
    adj_logits, adj_new, degp = _sample(z, u, gmax)
    yg = _h_pass(adj_new, x1, degp, b1.reshape(1, H), W2)
    nc_logits = _nc_pass(adj_new, yg, degp, b2.reshape(1, C))
    return (nc_logits, adj_logits)


# speedup vs baseline: 2.6850x; 1.3232x over previous
import jax
import jax.numpy as jnp
import numpy as np
from jax.experimental import pallas as pl
from jax.experimental.pallas import tpu as pltpu

N = 4096
D = 256
H = 128
EMB = 64
C = 40
TEMP = 1.0
EPS = 1e-06

BM = 512
BS = 1024
NB = N // BS

_ORDER = []
for _i in range(NB):
    _ORDER.append((_i, _i))
    for _j in range(_i + 1, NB):
        _ORDER.append((_i, _j))
        _ORDER.append((_j, _i))
_OI = np.array([p[0] for p in _ORDER], dtype=np.int32)
_OJ = np.array([p[1] for p in _ORDER], dtype=np.int32)
_RC = np.array(
    [1] + [int((min(a), max(a)) != (min(b), max(b)))
           for a, b in zip(_ORDER[1:], _ORDER[:-1])], dtype=np.int32)

_K0 = np.uint32(0)
_K1 = np.uint32(1234)
_K2 = np.uint32(_K0 ^ _K1 ^ np.uint32(0x1BD11BDA))
_KS = (_K0, _K1, _K2)
_ROT = ((13, 15, 26, 6), (17, 29, 16, 24))
_MINV = np.float32(EPS)
_SCALE = np.float32(np.float32(1.0 - EPS) - np.float32(EPS))


def _hidden_pass(features, wcat, adj):
    return pl.pallas_call(
        _hidden_kernel,
        grid=(N // BM,),
        in_specs=[
            pl.BlockSpec((N, 2 * H), lambda i: (0, 0)),
            pl.BlockSpec((D, 2 * H), lambda i: (0, 0)),
            pl.BlockSpec((BM, N), lambda i: (i, 0)),
        ],
        out_specs=[
            pl.BlockSpec((BM, H), lambda i: (i, 0)),
            pl.BlockSpec((N, H), lambda i: (0, 0)),
        ],
        out_shape=[
            jax.ShapeDtypeStruct((N, H), jnp.float32),
            jax.ShapeDtypeStruct((N, H), jnp.float32),
        ],
        scratch_shapes=[pltpu.VMEM((N, 2 * H), jnp.float32)],
    )(features, wcat, adj)


def _hidden_kernel(feat_ref, wcat_ref, a_ref, hid_ref, x1_ref, xw_ref):
    @pl.when(pl.program_id(0) == 0)
    def _():
        xw_ref[...] = jnp.dot(feat_ref[...], wcat_ref[...])
        x1_ref[...] = xw_ref[:, H:]

    hid_ref[...] = jnp.dot(a_ref[...], xw_ref[:, :H])


def _z_kernel(hid_ref, wm_ref, a_ref, z_ref, gmax_ref, m_ref):
    @pl.when(pl.program_id(0) == 0)
    def _():
        m_ref[...] = jnp.dot(hid_ref[...], wm_ref[...])

    z = jnp.maximum(jnp.dot(a_ref[...], m_ref[...]), 0.0)
    z_ref[...] = z
    bm = jnp.max(jnp.sum(z * z, axis=1)).reshape(1, 1)

    @pl.when(pl.program_id(0) == 0)
    def _():
        gmax_ref[...] = bm

    @pl.when(pl.program_id(0) > 0)
    def _():
        gmax_ref[...] = jnp.maximum(gmax_ref[...], bm)


def _z_pass(hidden, wm, adj):
    return pl.pallas_call(
        _z_kernel,
        grid=(N // BM,),
        in_specs=[
            pl.BlockSpec((N, H), lambda i: (0, 0)),
            pl.BlockSpec((H, EMB), lambda i: (0, 0)),
            pl.BlockSpec((BM, N), lambda i: (i, 0)),
        ],
        out_specs=[
            pl.BlockSpec((BM, EMB), lambda i: (i, 0)),
            pl.BlockSpec((1, 1), lambda i: (0, 0)),
        ],
        out_shape=[
            jax.ShapeDtypeStruct((N, EMB), jnp.float32),
            jax.ShapeDtypeStruct((1, 1), jnp.float32),
        ],
        scratch_shapes=[pltpu.VMEM((N, EMB), jnp.float32)],
    )(hidden, wm, adj)


def _sample_kernel(oi_ref, oj_ref, rc_ref, zi_ref, zj_ref, gmax_ref,
                   logits_ref, adj_ref, degp_ref, ln_ref):
    t = pl.program_id(0)
    i = oi_ref[t]
    j = oj_ref[t]

    @pl.when(rc_ref[t] == 1)
    def _():
        a = jnp.minimum(i, j)
        b = jnp.maximum(i, j)
        rr = a * BS + jax.lax.broadcasted_iota(jnp.int32, (BS, BS), 0)
        cc = b * BS + jax.lax.broadcasted_iota(jnp.int32, (BS, BS), 1)
        x1 = (rr * N + cc).astype(jnp.uint32)
        x0 = jnp.zeros_like(x1)
        x0 = x0 + _KS[0]
        x1 = x1 + _KS[1]
        for it in range(5):
            for r in _ROT[it % 2]:
                x0 = x0 + x1
                x1 = (x1 << np.uint32(r)) | (x1 >> np.uint32(32 - r))
                x1 = x0 ^ x1
            x0 = x0 + _KS[(it + 1) % 3]
            x1 = x1 + _KS[(it + 2) % 3] + np.uint32(it + 1)
        bits = x0 ^ x1
        fb = (bits >> np.uint32(9)) | np.uint32(0x3F800000)
        f = jax.lax.bitcast_convert_type(fb, jnp.float32) - 1.0
        u = jnp.maximum(_MINV, f * _SCALE + _MINV)
        ln_ref[...] = jnp.log(u) - jnp.log1p(-u)

    l = jax.lax.dot_general(zi_ref[...], zj_ref[...], (((1,), (1,)), ((), ())))
    logits_ref[...] = l
    gmax = gmax_ref[0, 0]
    p = jnp.clip(l / gmax, EPS, 1.0 - EPS)
    lp = jnp.log(p) - jnp.log1p(-p)
    rows = i * BS + jax.lax.broadcasted_iota(jnp.int32, (BS, BS), 0)
    cols = j * BS + jax.lax.broadcasted_iota(jnp.int32, (BS, BS), 1)
    ln_blk = ln_ref[...]
    ln = jnp.where(rows < cols, ln_blk, ln_blk.T)
    soft = jax.nn.sigmoid((lp + ln) / TEMP)
    hard = jnp.round(soft)
    a = jnp.where(rows == cols, 0.0, hard)
    adj_ref[...] = a.astype(jnp.int8)
    degp_ref[...] = jnp.sum(a, axis=1, keepdims=True).reshape(1, BS, 1)


def _sample(z, gmax):
    grid_spec = pltpu.PrefetchScalarGridSpec(
        num_scalar_prefetch=3,
        grid=(len(_ORDER),),
        in_specs=[
            pl.BlockSpec((BS, EMB), lambda t, oi, oj, rc: (oi[t], 0)),
            pl.BlockSpec((BS, EMB), lambda t, oi, oj, rc: (oj[t], 0)),
            pl.BlockSpec((1, 1), lambda t, oi, oj, rc: (0, 0)),
        ],
        out_specs=[
            pl.BlockSpec((BS, BS), lambda t, oi, oj, rc: (oi[t], oj[t])),
            pl.BlockSpec((BS, BS), lambda t, oi, oj, rc: (oi[t], oj[t])),
            pl.BlockSpec((1, BS, 1), lambda t, oi, oj, rc: (oj[t], oi[t], 0)),
        ],
        scratch_shapes=[pltpu.VMEM((BS, BS), jnp.float32)],
    )
    return pl.pallas_call(
        _sample_kernel,
        grid_spec=grid_spec,
        out_shape=[
            jax.ShapeDtypeStruct((N, N), jnp.float32),
            jax.ShapeDtypeStruct((N, N), jnp.int8),
            jax.ShapeDtypeStruct((NB, N, 1), jnp.float32),
        ],
    )(jnp.asarray(_OI), jnp.asarray(_OJ), jnp.asarray(_RC), z, z, gmax)


def _h_kernel(a_ref, x1_ref, degp_ref, b1_ref, w2_ref, yg_ref):
    i = pl.program_id(0)
    deg = jnp.sum(degp_ref[...], axis=0) + 1.0
    dinv = 1.0 / jnp.sqrt(deg)
    y1 = dinv * x1_ref[...]
    a = a_ref[...].astype(jnp.bfloat16)
    acc = jnp.dot(a, y1.astype(jnp.bfloat16),
                  preferred_element_type=jnp.float32)
    deg_i = jnp.sum(degp_ref[:, pl.ds(i * BM, BM), :], axis=0) + 1.0
    dinv_i = 1.0 / jnp.sqrt(deg_i)
    y1_i = dinv_i * x1_ref[pl.ds(i * BM, BM), :]
    h = jnp.maximum(dinv_i * (acc + y1_i) + b1_ref[...], 0.0)
    yg_ref[...] = dinv_i * jnp.dot(h, w2_ref[...])


def _h_pass(adj_new, x1, degp, b1, w2):
    return pl.pallas_call(
        _h_kernel,
        grid=(N // BM,),
        in_specs=[
            pl.BlockSpec((BM, N), lambda i: (i, 0)),
            pl.BlockSpec((N, H), lambda i: (0, 0)),
            pl.BlockSpec((NB, N, 1), lambda i: (0, 0, 0)),
            pl.BlockSpec((1, H), lambda i: (0, 0)),
            pl.BlockSpec((H, C), lambda i: (0, 0)),
        ],
        out_specs=pl.BlockSpec((BM, C), lambda i: (i, 0)),
        out_shape=jax.ShapeDtypeStruct((N, C), jnp.float32),
    )(adj_new, x1, degp, b1, w2)


def _nc_kernel(a_ref, yg_ref, degp_ref, b2_ref, o_ref):
    i = pl.program_id(0)
    deg = jnp.sum(degp_ref[...], axis=0) + 1.0
    dinv = 1.0 / jnp.sqrt(deg)
    a = a_ref[...].astype(jnp.bfloat16)
    acc = jnp.dot(a, yg_ref[...].astype(jnp.bfloat16),
                  preferred_element_type=jnp.float32)
    yg_i = yg_ref[pl.ds(i * BM, BM), :]
    deg_i = jnp.sum(degp_ref[:, pl.ds(i * BM, BM), :], axis=0) + 1.0
    dinv_i = 1.0 / jnp.sqrt(deg_i)
    o_ref[...] = dinv_i * (acc + yg_i) + b2_ref[...]


def _nc_pass(adj_new, yg, degp, b2):
    return pl.pallas_call(
        _nc_kernel,
        grid=(N // BM,),
        in_specs=[
            pl.BlockSpec((BM, N), lambda i: (i, 0)),
            pl.BlockSpec((N, C), lambda i: (0, 0)),
            pl.BlockSpec((NB, N, 1), lambda i: (0, 0, 0)),
            pl.BlockSpec((1, C), lambda i: (0, 0)),
        ],
        out_specs=pl.BlockSpec((BM, C), lambda i: (i, 0)),
        out_shape=jax.ShapeDtypeStruct((N, C), jnp.float32),
    )(adj_new, yg, degp, b2)


def kernel(adj_norm, adj_orig, features, W_base, W_mean, W1, b1, W2, b2):
    wcat = jnp.concatenate([W_base, W1], axis=1)
    hidden, x1 = _hidden_pass(features, wcat, adj_norm)
    z, gmax = _z_pass(hidden, W_mean, adj_norm)
    adj_logits, adj_new, degp = _sample(z, gmax)
    yg = _h_pass(adj_new, x1, degp, b1.reshape(1, H), W2)
    nc_logits = _nc_pass(adj_new, yg, degp, b2.reshape(1, C))
    return (nc_logits, adj_logits)
